# Initial kernel scaffold; baseline (speedup 1.0000x reference)
import jax
import jax.numpy as jnp
from jax.experimental import pallas as pl


def kernel(inputs, table):
    raise NotImplementedError("write your pallas kernel here")


# speedup vs baseline: 3.4624x; 3.4624x over previous
import jax
import jax.numpy as jnp
from jax.experimental import pallas as pl


S_BLK = 512


def _bcast_kernel(table_ref, out_ref):
    out_ref[0] = table_ref[...]


def kernel(inputs, table):
    batch, seq = inputs.shape
    n_rows, d_model = table.shape
    grid = (seq // S_BLK, batch)
    return pl.pallas_call(
        _bcast_kernel,
        grid=grid,
        in_specs=[
            pl.BlockSpec((S_BLK, d_model), lambda i, b: (i, 0)),
        ],
        out_specs=pl.BlockSpec((1, S_BLK, d_model), lambda i, b: (b, i, 0)),
        out_shape=jax.ShapeDtypeStruct((batch, seq, d_model), table.dtype),
    )(table)


# speedup vs baseline: 5.8173x; 1.6801x over previous
import jax
import jax.numpy as jnp
from jax.experimental import pallas as pl


from jax.experimental.pallas import tpu as pltpu

S_BLK = 1024


def _bcast_kernel(table_ref, out_ref):
    out_ref[...] = jnp.broadcast_to(table_ref[...][None], out_ref.shape)


def kernel(inputs, table):
    batch, seq = inputs.shape
    n_rows, d_model = table.shape
    grid = (seq // S_BLK,)
    return pl.pallas_call(
        _bcast_kernel,
        grid=grid,
        in_specs=[
            pl.BlockSpec((S_BLK, d_model), lambda i: (i, 0)),
        ],
        out_specs=pl.BlockSpec((batch, S_BLK, d_model), lambda i: (0, i, 0)),
        out_shape=jax.ShapeDtypeStruct((batch, seq, d_model), table.dtype),
        compiler_params=pltpu.CompilerParams(
            dimension_semantics=("parallel",),
        ),
    )(table)
